# baseline (speedup 1.0000x reference)
import jax
import jax.numpy as jnp
from jax import lax
from jax.experimental import pallas as pl
from jax.experimental.pallas import tpu as pltpu
from jax.experimental.pallas import tpu_sc as plsc

N = 10000
H = 128
G = 64

_NC = 2
_NS = 16
_CHUNK = 128
_KMAX = -(-(320000 // _CHUNK // _NC) // _NS)

_DOT = dict(preferred_element_type=jnp.float32,
            precision=jax.lax.Precision.DEFAULT)


_NBUF = 3


def _seg_sum_body(msg_hbm, src_hbm, dst_hbm, out_hbm, sidx, didx,
                  rows0, rows1, rows2, acc,
                  sg0, sg1, sg2, si0, si1, si2):
    c = lax.axis_index("c")
    s = lax.axis_index("s")
    E = src_hbm.shape[0]
    n_chunks = E // _CHUNK
    per_core = n_chunks // _NC
    kmax = (per_core + _NS - 1) // _NS
    slab = (N // 8 // _NS) * 8
    chunks = []
    o = 0
    while o < slab:
        sz = min(_CHUNK, slab - o)
        chunks.append((o, sz))
        o += sz

    @pl.loop(0, _CHUNK)
    def _(r):
        @pl.loop(0, H // 16)
        def _(j):
            rows0[r, pl.ds(j * 16, 16)] = jnp.zeros((16,), jnp.float32)

    row0 = pl.multiple_of(s * slab, 8)
    for o, sz in chunks:
        pltpu.sync_copy(rows0.at[pl.ds(0, sz)],
                        acc.at[pl.ds(row0 + o, sz)])

    @pl.when(s == 0)
    def _():
        r0 = pl.multiple_of(slab * _NS, 8)
        pltpu.sync_copy(rows0.at[pl.ds(0, N - slab * _NS)],
                        acc.at[pl.ds(r0, N - slab * _NS)])

    base_k = per_core // _NS
    extra = per_core - base_k * _NS
    cs = c * per_core + s * base_k + jnp.minimum(s, extra)
    kn = base_k + jnp.where(s < extra, 1, 0)
    edge0 = cs * _CHUNK

    bufs = ((rows0, sg0, si0), (rows1, sg1, si1), (rows2, sg2, si2))

    def _idx_copies(k, b, si_b):
        return (pltpu.make_async_copy(
                    src_hbm.at[pl.ds(edge0 + k * _CHUNK, _CHUNK)],
                    sidx.at[b], si_b),
                pltpu.make_async_copy(
                    dst_hbm.at[pl.ds(edge0 + k * _CHUNK, _CHUNK)],
                    didx.at[b], si_b))

    def _gather_copy(k, b, rows_b, sg_b):
        return pltpu.make_async_copy(
            msg_hbm.at[sidx.at[b]], rows_b, sg_b)

    for b in range(_NBUF):
        rows_b, sg_b, si_b = bufs[b]

        @pl.when(b < kn)
        def _(b=b, si_b=si_b):
            for cp in _idx_copies(b, b, si_b):
                cp.start()

    for b in range(2):
        rows_b, sg_b, si_b = bufs[b]

        @pl.when(b < kn)
        def _(b=b, rows_b=rows_b, sg_b=sg_b, si_b=si_b):
            for cp in _idx_copies(b, b, si_b):
                cp.wait()
            _gather_copy(b, b, rows_b, sg_b).start()

    plsc.subcore_barrier()

    @pl.loop(0, (kmax + _NBUF - 1) // _NBUF)
    def _(kq):
        for b in range(_NBUF):
            rows_b, sg_b, si_b = bufs[b]
            b2 = (b + 2) % _NBUF
            rows_b2, sg_b2, si_b2 = bufs[b2]
            k = kq * _NBUF + b

            @pl.when(k < kn)
            def _(k=k, b=b, rows_b=rows_b, sg_b=sg_b, si_b=si_b,
                  b2=b2, rows_b2=rows_b2, sg_b2=sg_b2, si_b2=si_b2):
                _gather_copy(k, b, rows_b, sg_b).wait()
                pltpu.sync_copy(rows_b, acc.at[didx.at[b]], add=True)

                @pl.when(k + _NBUF < kn)
                def _():
                    for cp in _idx_copies(k + _NBUF, b, si_b):
                        cp.start()

                @pl.when(k + 2 < kn)
                def _():
                    for cp in _idx_copies(k + 2, b2, si_b2):
                        cp.wait()
                    _gather_copy(k + 2, b2, rows_b2, sg_b2).start()

    plsc.subcore_barrier()

    for o, sz in chunks:
        r1 = pl.multiple_of(row0 + o, 8)
        pltpu.sync_copy(acc.at[pl.ds(r1, sz)],
                        out_hbm.at[pl.ds(pl.multiple_of(c * N + r1, 8), sz)])

    @pl.when(s == 0)
    def _():
        r0 = pl.multiple_of(slab * _NS, 8)
        pltpu.sync_copy(
            acc.at[pl.ds(r0, N - slab * _NS)],
            out_hbm.at[pl.ds(pl.multiple_of(c * N + r0, 8), N - slab * _NS)])


def _seg_sum(msg, src, dst):
    mesh = plsc.VectorSubcoreMesh(core_axis_name="c", subcore_axis_name="s")
    f = pl.kernel(
        _seg_sum_body,
        out_type=jax.ShapeDtypeStruct((_NC * N, H), jnp.float32),
        mesh=mesh,
        scratch_types=[
            pltpu.VMEM((_NBUF, _CHUNK), jnp.int32),
            pltpu.VMEM((_NBUF, _CHUNK), jnp.int32),
            pltpu.VMEM((_CHUNK, H), jnp.float32),
            pltpu.VMEM((_CHUNK, H), jnp.float32),
            pltpu.VMEM((_CHUNK, H), jnp.float32),
            pltpu.VMEM_SHARED((N, H), jnp.float32),
            pltpu.SemaphoreType.DMA,
            pltpu.SemaphoreType.DMA,
            pltpu.SemaphoreType.DMA,
            pltpu.SemaphoreType.DMA,
            pltpu.SemaphoreType.DMA,
            pltpu.SemaphoreType.DMA,
        ],
    )
    return f(msg, src, dst)


_RB = 5000


def _gru_body(p0_ref, p1_ref, h_ref, w_ref, wih_ref, whh_ref, bih_ref,
              bhh_ref, o_ref):
    agg = p0_ref[...] + p1_ref[...]
    m = lax.dot_general(agg, w_ref[...], (((1,), (0,)), ((), ())), **_DOT)
    h = h_ref[...]
    gi = lax.dot_general(m, wih_ref[...], (((1,), (1,)), ((), ())), **_DOT)
    gi = gi + bih_ref[...][None, :]
    gh = lax.dot_general(h, whh_ref[...], (((1,), (1,)), ((), ())), **_DOT)
    gh = gh + bhh_ref[...][None, :]
    r = jax.nn.sigmoid(gi[:, 0:H] + gh[:, 0:H])
    z = jax.nn.sigmoid(gi[:, H:2 * H] + gh[:, H:2 * H])
    n = jnp.tanh(gi[:, 2 * H:3 * H] + r * gh[:, 2 * H:3 * H])
    o_ref[...] = (1.0 - z) * n + z * h


def _gru_call(parts, h, w, w_ih, w_hh, b_ih, b_hh):
    nb = N // _RB
    return pl.pallas_call(
        _gru_body,
        grid=(nb,),
        in_specs=[pl.BlockSpec((_RB, H), lambda i: (i, 0)),
                  pl.BlockSpec((_RB, H), lambda i, nb=nb: (i + nb, 0)),
                  pl.BlockSpec((_RB, H), lambda i: (i, 0)),
                  pl.BlockSpec((H, H), lambda i: (0, 0)),
                  pl.BlockSpec((3 * H, H), lambda i: (0, 0)),
                  pl.BlockSpec((3 * H, H), lambda i: (0, 0)),
                  pl.BlockSpec((3 * H,), lambda i: (0,)),
                  pl.BlockSpec((3 * H,), lambda i: (0,))],
        out_specs=pl.BlockSpec((_RB, H), lambda i: (i, 0)),
        out_shape=jax.ShapeDtypeStruct((N, H), jnp.float32),
    )(parts, parts, h, w, w_ih, w_hh, b_ih, b_hh)


def _gru_tail_body(p0_ref, p1_ref, h_ref, w_ref, wih_ref, whh_ref, bih_ref,
                   bhh_ref, w1_ref, b1_ref, bng_ref, bnb_ref, batch_ref,
                   w2_ref, b2_ref, o_ref, y_scr, st_scr):
    nb = N // _RB
    i = pl.program_id(0)

    @pl.when(i < nb)
    def _():
        agg = p0_ref[...] + p1_ref[...]
        m = lax.dot_general(agg, w_ref[...], (((1,), (0,)), ((), ())), **_DOT)
        h = h_ref[...]
        gi = lax.dot_general(m, wih_ref[...], (((1,), (1,)), ((), ())),
                             **_DOT) + bih_ref[...][None, :]
        gh = lax.dot_general(h, whh_ref[...], (((1,), (1,)), ((), ())),
                             **_DOT) + bhh_ref[...][None, :]
        r = jax.nn.sigmoid(gi[:, 0:H] + gh[:, 0:H])
        z = jax.nn.sigmoid(gi[:, H:2 * H] + gh[:, H:2 * H])
        n = jnp.tanh(gi[:, 2 * H:3 * H] + r * gh[:, 2 * H:3 * H])
        hn = (1.0 - z) * n + z * h
        y = lax.dot_general(hn, w1_ref[...], (((1,), (1,)), ((), ())),
                            **_DOT) + b1_ref[...][None, :]
        y_scr[pl.ds(pl.multiple_of(i * _RB, 8), _RB), :] = y
        st = jnp.stack([jnp.sum(y, axis=0), jnp.sum(y * y, axis=0)], axis=0)

        @pl.when(i == 0)
        def _():
            st_scr[...] = st

        @pl.when(i != 0)
        def _():
            st_scr[...] += st

    @pl.when(i == nb)
    def _():
        mean = st_scr[0, :] / N
        var = st_scr[1, :] / N - mean * mean
        scale = bng_ref[...] * lax.rsqrt(var + 1e-5)
        y = (y_scr[...] - mean[None, :]) * scale[None, :]
        y = jnp.maximum(y + bnb_ref[...][None, :], 0.0)
        b = batch_ref[0, :]
        onehot = (b[:, None] == lax.broadcasted_iota(jnp.int32, (N, G), 1))
        onehot = onehot.astype(jnp.float32)
        gs = lax.dot_general(onehot, y, (((0,), (0,)), ((), ())), **_DOT)
        gc = jnp.sum(onehot, axis=0)
        gm = gs / jnp.maximum(gc, 1.0)[:, None]
        logits = lax.dot_general(gm, w2_ref[...], (((1,), (1,)), ((), ())),
                                 **_DOT) + b2_ref[...][None, :]
        mx = jnp.max(logits, axis=-1, keepdims=True)
        sh = logits - mx
        lse = jnp.log(jnp.sum(jnp.exp(sh), axis=-1, keepdims=True))
        o_ref[...] = sh - lse


def _gru_tail_call(parts, h, w, w_ih, w_hh, b_ih, b_hh,
                   fc1_w, fc1_b, bn_g, bn_b, batch, fc2_w, fc2_b):
    C = fc2_w.shape[0]
    nb = N // _RB
    blk = lambda i: (jnp.minimum(i, nb - 1), 0)
    blk2 = lambda i: (jnp.minimum(i, nb - 1) + nb, 0)
    full = lambda i: (0, 0)
    vec = lambda i: (0,)
    return pl.pallas_call(
        _gru_tail_body,
        grid=(nb + 1,),
        in_specs=[pl.BlockSpec((_RB, H), blk),
                  pl.BlockSpec((_RB, H), blk2),
                  pl.BlockSpec((_RB, H), blk),
                  pl.BlockSpec((H, H), full),
                  pl.BlockSpec((3 * H, H), full),
                  pl.BlockSpec((3 * H, H), full),
                  pl.BlockSpec((3 * H,), vec),
                  pl.BlockSpec((3 * H,), vec),
                  pl.BlockSpec((H, H), full),
                  pl.BlockSpec((H,), vec),
                  pl.BlockSpec((H,), vec),
                  pl.BlockSpec((H,), vec),
                  pl.BlockSpec((1, N), full),
                  pl.BlockSpec((C, H), full),
                  pl.BlockSpec((C,), vec)],
        out_specs=pl.BlockSpec((G, C), full),
        out_shape=jax.ShapeDtypeStruct((G, C), jnp.float32),
        scratch_shapes=[pltpu.VMEM((N, H), jnp.float32),
                        pltpu.VMEM((2, H), jnp.float32)],
    )(parts, parts, h, w, w_ih, w_hh, b_ih, b_hh,
      fc1_w, fc1_b, bn_g, bn_b, batch.reshape(1, N), fc2_w, fc2_b)


def kernel(x, edge_index, batch, weight, w_ih, w_hh, b_ih, b_hh,
           fc1_w, fc1_b, bn_g, bn_b, fc2_w, fc2_b):
    src = edge_index[0]
    dst = edge_index[1]
    h = x
    L = weight.shape[0]
    for l in range(L - 1):
        parts = _seg_sum(h, src, dst)
        h = _gru_call(parts, h, weight[l], w_ih, w_hh, b_ih, b_hh)
    parts = _seg_sum(h, src, dst)
    return _gru_tail_call(parts, h, weight[L - 1], w_ih, w_hh, b_ih, b_hh,
                          fc1_w, fc1_b, bn_g, bn_b, batch, fc2_w, fc2_b)


# speedup vs baseline: 1.1007x; 1.1007x over previous
import jax
import jax.numpy as jnp
from jax import lax
from jax.experimental import pallas as pl
from jax.experimental.pallas import tpu as pltpu
from jax.experimental.pallas import tpu_sc as plsc

N = 10000
H = 128
G = 64

_NC = 2
_NS = 16
_CHUNK = 128
_KMAX = -(-(320000 // _CHUNK // _NC) // _NS)

_DOT = dict(preferred_element_type=jnp.float32,
            precision=jax.lax.Precision.DEFAULT)


_NBUF = 3


def _seg_sum_body(msg_hbm, src_hbm, dst_hbm, out_hbm, sidx, didx,
                  rows0, rows1, rows2, acc,
                  sg0, sg1, sg2, si0, si1, si2, ss0, ss1, ss2):
    c = lax.axis_index("c")
    s = lax.axis_index("s")
    E = src_hbm.shape[0]
    n_chunks = E // _CHUNK
    per_core = n_chunks // _NC
    kmax = (per_core + _NS - 1) // _NS
    slab = (N // 8 // _NS) * 8
    chunks = []
    o = 0
    while o < slab:
        sz = min(_CHUNK, slab - o)
        chunks.append((o, sz))
        o += sz

    @pl.loop(0, _CHUNK)
    def _(r):
        @pl.loop(0, H // 16)
        def _(j):
            rows0[r, pl.ds(j * 16, 16)] = jnp.zeros((16,), jnp.float32)

    row0 = pl.multiple_of(s * slab, 8)
    for o, sz in chunks:
        pltpu.sync_copy(rows0.at[pl.ds(0, sz)],
                        acc.at[pl.ds(row0 + o, sz)])

    @pl.when(s == 0)
    def _():
        r0 = pl.multiple_of(slab * _NS, 8)
        pltpu.sync_copy(rows0.at[pl.ds(0, N - slab * _NS)],
                        acc.at[pl.ds(r0, N - slab * _NS)])

    base_k = per_core // _NS
    extra = per_core - base_k * _NS
    cs = c * per_core + s * base_k + jnp.minimum(s, extra)
    kn = base_k + jnp.where(s < extra, 1, 0)
    edge0 = cs * _CHUNK

    bufs = ((rows0, sg0, si0, ss0), (rows1, sg1, si1, ss1),
            (rows2, sg2, si2, ss2))

    def _idx_copies(k, b, si_b):
        return (pltpu.make_async_copy(
                    src_hbm.at[pl.ds(edge0 + k * _CHUNK, _CHUNK)],
                    sidx.at[b], si_b),
                pltpu.make_async_copy(
                    dst_hbm.at[pl.ds(edge0 + k * _CHUNK, _CHUNK)],
                    didx.at[lax.rem(k, 2 * _NBUF)], si_b))

    def _gather_copy(k, b, rows_b, sg_b):
        return pltpu.make_async_copy(
            msg_hbm.at[sidx.at[b]], rows_b, sg_b)

    def _scatter_start(k, rows_b, ss_b):
        pltpu.async_copy(
            rows_b, acc.at[didx.at[lax.rem(k, 2 * _NBUF)]], ss_b, add=True)

    def _scatter_wait(k, rows_b, ss_b):
        pltpu.make_async_copy(
            rows_b, acc.at[didx.at[lax.rem(k, 2 * _NBUF)]], ss_b).wait()

    for b in range(_NBUF):
        rows_b, sg_b, si_b, ss_b = bufs[b]

        @pl.when(b < kn)
        def _(b=b, si_b=si_b):
            for cp in _idx_copies(b, b, si_b):
                cp.start()

    for b in range(2):
        rows_b, sg_b, si_b, ss_b = bufs[b]

        @pl.when(b < kn)
        def _(b=b, rows_b=rows_b, sg_b=sg_b, si_b=si_b):
            for cp in _idx_copies(b, b, si_b):
                cp.wait()
            _gather_copy(b, b, rows_b, sg_b).start()

    plsc.subcore_barrier()

    @pl.loop(0, (kmax + _NBUF) // _NBUF)
    def _(kq):
        for b in range(_NBUF):
            rows_b, sg_b, si_b, ss_b = bufs[b]
            b1 = (b + _NBUF - 1) % _NBUF
            rows_b1, sg_b1, si_b1, ss_b1 = bufs[b1]
            b2 = (b + 2) % _NBUF
            rows_b2, sg_b2, si_b2, ss_b2 = bufs[b2]
            k = kq * _NBUF + b

            @pl.when((k >= 1) & (k <= kn))
            def _(k=k, rows_b1=rows_b1, ss_b1=ss_b1):
                _scatter_wait(k - 1, rows_b1, ss_b1)

            @pl.when(k < kn)
            def _(k=k, b=b, rows_b=rows_b, sg_b=sg_b, si_b=si_b, ss_b=ss_b,
                  b2=b2, rows_b2=rows_b2, sg_b2=sg_b2, si_b2=si_b2):
                _gather_copy(k, b, rows_b, sg_b).wait()
                _scatter_start(k, rows_b, ss_b)

                @pl.when(k + _NBUF < kn)
                def _():
                    for cp in _idx_copies(k + _NBUF, b, si_b):
                        cp.start()

                @pl.when(k + 2 < kn)
                def _():
                    for cp in _idx_copies(k + 2, b2, si_b2):
                        cp.wait()
                    _gather_copy(k + 2, b2, rows_b2, sg_b2).start()

    plsc.subcore_barrier()

    for o, sz in chunks:
        r1 = pl.multiple_of(row0 + o, 8)
        pltpu.sync_copy(acc.at[pl.ds(r1, sz)],
                        out_hbm.at[pl.ds(pl.multiple_of(c * N + r1, 8), sz)])

    @pl.when(s == 0)
    def _():
        r0 = pl.multiple_of(slab * _NS, 8)
        pltpu.sync_copy(
            acc.at[pl.ds(r0, N - slab * _NS)],
            out_hbm.at[pl.ds(pl.multiple_of(c * N + r0, 8), N - slab * _NS)])


def _seg_sum(msg, src, dst):
    mesh = plsc.VectorSubcoreMesh(core_axis_name="c", subcore_axis_name="s")
    f = pl.kernel(
        _seg_sum_body,
        out_type=jax.ShapeDtypeStruct((_NC * N, H), jnp.float32),
        mesh=mesh,
        scratch_types=[
            pltpu.VMEM((_NBUF, _CHUNK), jnp.int32),
            pltpu.VMEM((2 * _NBUF, _CHUNK), jnp.int32),
            pltpu.VMEM((_CHUNK, H), jnp.float32),
            pltpu.VMEM((_CHUNK, H), jnp.float32),
            pltpu.VMEM((_CHUNK, H), jnp.float32),
            pltpu.VMEM_SHARED((N, H), jnp.float32),
            pltpu.SemaphoreType.DMA,
            pltpu.SemaphoreType.DMA,
            pltpu.SemaphoreType.DMA,
            pltpu.SemaphoreType.DMA,
            pltpu.SemaphoreType.DMA,
            pltpu.SemaphoreType.DMA,
            pltpu.SemaphoreType.DMA,
            pltpu.SemaphoreType.DMA,
            pltpu.SemaphoreType.DMA,
        ],
    )
    return f(msg, src, dst)


_RB = 2000


def _gru_body(p0_ref, p1_ref, h_ref, w_ref, wih_ref, whh_ref, bih_ref,
              bhh_ref, o_ref):
    agg = p0_ref[...] + p1_ref[...]
    m = lax.dot_general(agg, w_ref[...], (((1,), (0,)), ((), ())), **_DOT)
    h = h_ref[...]
    gi = lax.dot_general(m, wih_ref[...], (((1,), (1,)), ((), ())), **_DOT)
    gi = gi + bih_ref[...][None, :]
    gh = lax.dot_general(h, whh_ref[...], (((1,), (1,)), ((), ())), **_DOT)
    gh = gh + bhh_ref[...][None, :]
    r = jax.nn.sigmoid(gi[:, 0:H] + gh[:, 0:H])
    z = jax.nn.sigmoid(gi[:, H:2 * H] + gh[:, H:2 * H])
    n = jnp.tanh(gi[:, 2 * H:3 * H] + r * gh[:, 2 * H:3 * H])
    o_ref[...] = (1.0 - z) * n + z * h


def _gru_call(parts, h, w, w_ih, w_hh, b_ih, b_hh):
    nb = N // _RB
    return pl.pallas_call(
        _gru_body,
        grid=(nb,),
        in_specs=[pl.BlockSpec((_RB, H), lambda i: (i, 0)),
                  pl.BlockSpec((_RB, H), lambda i, nb=nb: (i + nb, 0)),
                  pl.BlockSpec((_RB, H), lambda i: (i, 0)),
                  pl.BlockSpec((H, H), lambda i: (0, 0)),
                  pl.BlockSpec((3 * H, H), lambda i: (0, 0)),
                  pl.BlockSpec((3 * H, H), lambda i: (0, 0)),
                  pl.BlockSpec((3 * H,), lambda i: (0,)),
                  pl.BlockSpec((3 * H,), lambda i: (0,))],
        out_specs=pl.BlockSpec((_RB, H), lambda i: (i, 0)),
        out_shape=jax.ShapeDtypeStruct((N, H), jnp.float32),
    )(parts, parts, h, w, w_ih, w_hh, b_ih, b_hh)


def _gru_tail_body(p0_ref, p1_ref, h_ref, w_ref, wih_ref, whh_ref, bih_ref,
                   bhh_ref, w1_ref, b1_ref, bng_ref, bnb_ref, batch_ref,
                   w2_ref, b2_ref, o_ref, y_scr, st_scr):
    nb = N // _RB
    i = pl.program_id(0)

    @pl.when(i < nb)
    def _():
        agg = p0_ref[...] + p1_ref[...]
        m = lax.dot_general(agg, w_ref[...], (((1,), (0,)), ((), ())), **_DOT)
        h = h_ref[...]
        gi = lax.dot_general(m, wih_ref[...], (((1,), (1,)), ((), ())),
                             **_DOT) + bih_ref[...][None, :]
        gh = lax.dot_general(h, whh_ref[...], (((1,), (1,)), ((), ())),
                             **_DOT) + bhh_ref[...][None, :]
        r = jax.nn.sigmoid(gi[:, 0:H] + gh[:, 0:H])
        z = jax.nn.sigmoid(gi[:, H:2 * H] + gh[:, H:2 * H])
        n = jnp.tanh(gi[:, 2 * H:3 * H] + r * gh[:, 2 * H:3 * H])
        hn = (1.0 - z) * n + z * h
        y = lax.dot_general(hn, w1_ref[...], (((1,), (1,)), ((), ())),
                            **_DOT) + b1_ref[...][None, :]
        y_scr[pl.ds(pl.multiple_of(i * _RB, 8), _RB), :] = y
        st = jnp.stack([jnp.sum(y, axis=0), jnp.sum(y * y, axis=0)], axis=0)

        @pl.when(i == 0)
        def _():
            st_scr[...] = st

        @pl.when(i != 0)
        def _():
            st_scr[...] += st

    @pl.when(i == nb)
    def _():
        mean = st_scr[0, :] / N
        var = st_scr[1, :] / N - mean * mean
        scale = bng_ref[...] * lax.rsqrt(var + 1e-5)
        y = (y_scr[...] - mean[None, :]) * scale[None, :]
        y = jnp.maximum(y + bnb_ref[...][None, :], 0.0)
        b = batch_ref[0, :]
        onehot = (b[:, None] == lax.broadcasted_iota(jnp.int32, (N, G), 1))
        onehot = onehot.astype(jnp.float32)
        gs = lax.dot_general(onehot, y, (((0,), (0,)), ((), ())), **_DOT)
        gc = jnp.sum(onehot, axis=0)
        gm = gs / jnp.maximum(gc, 1.0)[:, None]
        logits = lax.dot_general(gm, w2_ref[...], (((1,), (1,)), ((), ())),
                                 **_DOT) + b2_ref[...][None, :]
        mx = jnp.max(logits, axis=-1, keepdims=True)
        sh = logits - mx
        lse = jnp.log(jnp.sum(jnp.exp(sh), axis=-1, keepdims=True))
        o_ref[...] = sh - lse


def _gru_tail_call(parts, h, w, w_ih, w_hh, b_ih, b_hh,
                   fc1_w, fc1_b, bn_g, bn_b, batch, fc2_w, fc2_b):
    C = fc2_w.shape[0]
    nb = N // _RB
    blk = lambda i: (jnp.minimum(i, nb - 1), 0)
    blk2 = lambda i: (jnp.minimum(i, nb - 1) + nb, 0)
    full = lambda i: (0, 0)
    vec = lambda i: (0,)
    return pl.pallas_call(
        _gru_tail_body,
        grid=(nb + 1,),
        in_specs=[pl.BlockSpec((_RB, H), blk),
                  pl.BlockSpec((_RB, H), blk2),
                  pl.BlockSpec((_RB, H), blk),
                  pl.BlockSpec((H, H), full),
                  pl.BlockSpec((3 * H, H), full),
                  pl.BlockSpec((3 * H, H), full),
                  pl.BlockSpec((3 * H,), vec),
                  pl.BlockSpec((3 * H,), vec),
                  pl.BlockSpec((H, H), full),
                  pl.BlockSpec((H,), vec),
                  pl.BlockSpec((H,), vec),
                  pl.BlockSpec((H,), vec),
                  pl.BlockSpec((1, N), full),
                  pl.BlockSpec((C, H), full),
                  pl.BlockSpec((C,), vec)],
        out_specs=pl.BlockSpec((G, C), full),
        out_shape=jax.ShapeDtypeStruct((G, C), jnp.float32),
        scratch_shapes=[pltpu.VMEM((N, H), jnp.float32),
                        pltpu.VMEM((2, H), jnp.float32)],
    )(parts, parts, h, w, w_ih, w_hh, b_ih, b_hh,
      fc1_w, fc1_b, bn_g, bn_b, batch.reshape(1, N), fc2_w, fc2_b)


def kernel(x, edge_index, batch, weight, w_ih, w_hh, b_ih, b_hh,
           fc1_w, fc1_b, bn_g, bn_b, fc2_w, fc2_b):
    src = edge_index[0]
    dst = edge_index[1]
    h = x
    L = weight.shape[0]
    for l in range(L - 1):
        parts = _seg_sum(h, src, dst)
        h = _gru_call(parts, h, weight[l], w_ih, w_hh, b_ih, b_hh)
    parts = _seg_sum(h, src, dst)
    return _gru_tail_call(parts, h, weight[L - 1], w_ih, w_hh, b_ih, b_hh,
                          fc1_w, fc1_b, bn_g, bn_b, batch, fc2_w, fc2_b)
